# Initial kernel scaffold; baseline (speedup 1.0000x reference)
import jax
import jax.numpy as jnp
from jax.experimental import pallas as pl


def kernel(x, edge_index, edge_weight, W_enc, b_enc, W_dec, b_dec):
    raise NotImplementedError("write your pallas kernel here")


# speedup vs baseline: 3.1472x; 3.1472x over previous
import functools

import jax
import jax.numpy as jnp
from jax import lax
from jax.experimental import pallas as pl
from jax.experimental.pallas import tpu as pltpu
from jax.experimental.pallas import tpu_sc as plsc

N = 10000
D = 128
E = 320000
H = 32

NC = 2
NS = 16
L = 16
NW = NC * NS

CHUNK = 1024
R = CHUNK // 128
CPW = 10
EPW = CHUNK * CPW
E_PAD = EPW * NW


def _enc_body(x_ref, w_ref, b_ref, y_ref):
    y_ref[...] = jnp.tanh(
        jnp.dot(x_ref[...], w_ref[...], preferred_element_type=jnp.float32)
        + b_ref[...]
    )


def _encode(x, W_enc, b_enc):
    return pl.pallas_call(
        _enc_body,
        out_shape=jax.ShapeDtypeStruct((N, H), jnp.float32),
    )(x, W_enc, b_enc.reshape(1, H))


def _rsqrt16(x):
    i = plsc.bitcast(x, jnp.int32)
    i = jnp.int32(0x5F3759DF) - (i >> 1)
    y = plsc.bitcast(i, jnp.float32)
    for _ in range(3):
        y = y * (jnp.float32(1.5) - jnp.float32(0.5) * x * y * y)
    return y


_mesh = plsc.VectorSubcoreMesh(core_axis_name="c", subcore_axis_name="s")


@functools.partial(
    pl.kernel,
    out_type=jax.ShapeDtypeStruct((NW, L), jnp.float32),
    mesh=_mesh,
    compiler_params=pltpu.CompilerParams(
        needs_layout_passes=False, use_tc_tiling_on_sc=False),
    scratch_types=[
        pltpu.VMEM((R, 128), jnp.int32),
        pltpu.VMEM((R, 128), jnp.int32),
        pltpu.VMEM((CHUNK,), jnp.float32),
        pltpu.VMEM((CHUNK, H), jnp.float32),
        pltpu.VMEM((CHUNK, H), jnp.float32),
        pltpu.VMEM((L,), jnp.float32),
        pltpu.SemaphoreType.DMA,
    ],
)
def _sc_edge_partials(row_hbm, col_hbm, w_hbm, y_hbm, out_hbm,
                      idxr_v, idxc_v, w_v, gr_v, gc_v, acc_v, sem):
    wid = lax.axis_index("s") * NC + lax.axis_index("c")

    def chunk_body(c, acc):
        ebase = wid * EPW + c * CHUNK
        rbase = wid * (EPW // 128) + c * R
        pltpu.sync_copy(row_hbm.at[pl.ds(rbase, R)], idxr_v)
        pltpu.sync_copy(col_hbm.at[pl.ds(rbase, R)], idxc_v)
        pltpu.sync_copy(w_hbm.at[pl.ds(ebase, CHUNK)], w_v)
        cps = []
        for r in range(R):
            cps.append(pltpu.async_copy(
                y_hbm.at[idxr_v.at[r]], gr_v.at[pl.ds(r * 128, 128)], sem))
            cps.append(pltpu.async_copy(
                y_hbm.at[idxc_v.at[r]], gc_v.at[pl.ds(r * 128, 128)], sem))
        for cp in cps:
            cp.wait()

        def e_body(e16, acc_in):
            lane = e16 * L + lax.iota(jnp.int32, L)
            s = jnp.zeros((L,), jnp.float32)
            for f in range(H):
                fv = jnp.full((L,), f, jnp.int32)
                g1 = plsc.load_gather(gr_v, [lane, fv])
                g2 = plsc.load_gather(gc_v, [lane, fv])
                d = g1 - g2
                s = s + d * d
            s = s + jnp.float32(1e-12)
            dist = s * _rsqrt16(s)
            wv = w_v[pl.ds(e16 * L, L)]
            return acc_in + dist * wv

        return lax.fori_loop(0, CHUNK // L, e_body, acc)

    acc = lax.fori_loop(0, CPW, chunk_body, jnp.zeros((L,), jnp.float32))
    acc_v[...] = acc
    pltpu.sync_copy(acc_v, out_hbm.at[wid])


def _comb_body(x_ref, y_ref, w_ref, b_ref, p_ref, o_ref):
    xh = (jnp.dot(y_ref[...], w_ref[...], preferred_element_type=jnp.float32)
          + b_ref[...])
    r = x_ref[...] - xh
    l1 = jnp.sum(r * r) * jnp.float32(1.0 / (N * D))
    l2 = jnp.sum(p_ref[...]) * jnp.float32(1.0 / E)
    o_ref[0, 0] = l1 + l2


def _combine(x, Y, W_dec, b_dec, parts):
    return pl.pallas_call(
        _comb_body,
        out_shape=jax.ShapeDtypeStruct((1, 1), jnp.float32),
        out_specs=pl.BlockSpec(memory_space=pltpu.SMEM),
    )(x, Y, W_dec, b_dec.reshape(1, D), parts)


def kernel(x, edge_index, edge_weight, W_enc, b_enc, W_dec, b_dec):
    pad = E_PAD - E
    row = jnp.concatenate([edge_index[0], jnp.zeros((pad,), jnp.int32)])
    col = jnp.concatenate([edge_index[1], jnp.zeros((pad,), jnp.int32)])
    w = jnp.concatenate([edge_weight, jnp.zeros((pad,), jnp.float32)])
    row2d = row.reshape(E_PAD // 128, 128)
    col2d = col.reshape(E_PAD // 128, 128)

    Y = _encode(x, W_enc, b_enc)
    parts = _sc_edge_partials(row2d, col2d, w, Y)
    out = _combine(x, Y, W_dec, b_dec, parts)
    return out[0, 0]


# speedup vs baseline: 12.5547x; 3.9892x over previous
import functools

import jax
import jax.numpy as jnp
from jax import lax
from jax.experimental import pallas as pl
from jax.experimental.pallas import tpu as pltpu
from jax.experimental.pallas import tpu_sc as plsc

N = 10000
D = 128
E = 320000
H = 32
HP = H // 2

NC = 2
NS = 16
L = 16
NW = NC * NS

CHUNK = 1024
R = CHUNK // 128
CPW = 10
EPW = CHUNK * CPW
E_PAD = EPW * NW
RPW = EPW // 128


def _enc_body(x_ref, w_ref, b_ref, y_ref):
    y_ref[...] = jnp.tanh(
        jnp.dot(x_ref[...], w_ref[...], preferred_element_type=jnp.float32)
        + b_ref[...]
    )


def _encode(x, W_enc, b_enc):
    return pl.pallas_call(
        _enc_body,
        out_shape=jax.ShapeDtypeStruct((N, H), jnp.float32),
    )(x, W_enc, b_enc.reshape(1, H))


def _sqrt16(x):
    i = plsc.bitcast(x, jnp.int32)
    i = jnp.int32(0x5F3759DF) - (i >> 1)
    y = plsc.bitcast(i, jnp.float32)
    for _ in range(3):
        y = y * (jnp.float32(1.5) - jnp.float32(0.5) * x * y * y)
    return x * y


def _halves(acc_bf):
    ai = plsc.bitcast(acc_bf, jnp.int32)
    lo = plsc.bitcast(lax.shift_left(ai, 16), jnp.float32)
    hi = plsc.bitcast(jnp.bitwise_and(ai, jnp.int32(-65536)), jnp.float32)
    return lo + hi


_mesh = plsc.VectorSubcoreMesh(core_axis_name="c", subcore_axis_name="s")


@functools.partial(
    pl.kernel,
    out_type=jax.ShapeDtypeStruct((NW, L), jnp.float32),
    mesh=_mesh,
    compiler_params=pltpu.CompilerParams(
        needs_layout_passes=False, use_tc_tiling_on_sc=False),
    scratch_types=[
        pltpu.VMEM_SHARED((N, HP), jnp.int32),
        pltpu.VMEM((RPW, 128), jnp.int32),
        pltpu.VMEM((RPW, 128), jnp.int32),
        pltpu.VMEM((EPW,), jnp.float32),
        pltpu.VMEM((CHUNK, HP), jnp.int32),
        pltpu.VMEM((CHUNK, HP), jnp.int32),
        pltpu.VMEM((CHUNK, HP), jnp.int32),
        pltpu.VMEM((CHUNK, HP), jnp.int32),
        pltpu.VMEM((L,), jnp.float32),
        pltpu.SemaphoreType.DMA,
        pltpu.SemaphoreType.DMA,
    ],
)
def _sc_edge_partials(row_hbm, col_hbm, w_hbm, yp_hbm, out_hbm,
                      ysh, idxr_v, idxc_v, w_v,
                      gra_v, gca_v, grb_v, gcb_v, acc_v, sema, semb):
    cid = lax.axis_index("c")
    sid = lax.axis_index("s")
    wid = sid * NC + cid

    @pl.when(sid == 0)
    def _stage():
        pltpu.sync_copy(yp_hbm, ysh)

    pltpu.sync_copy(row_hbm.at[pl.ds(wid * RPW, RPW)], idxr_v)
    pltpu.sync_copy(col_hbm.at[pl.ds(wid * RPW, RPW)], idxc_v)
    pltpu.sync_copy(w_hbm.at[pl.ds(wid * EPW, EPW)], w_v)
    plsc.subcore_barrier()

    bufs = [(gra_v, gca_v, sema), (grb_v, gcb_v, semb)]

    def fire(c):
        gr, gc, sem = bufs[c % 2]
        cps = []
        for r in range(R):
            cps.append(pltpu.async_copy(
                ysh.at[idxr_v.at[c * R + r]],
                gr.at[pl.ds(r * 128, 128)], sem))
            cps.append(pltpu.async_copy(
                ysh.at[idxc_v.at[c * R + r]],
                gc.at[pl.ds(r * 128, 128)], sem))
        return cps

    def compute(c, acc):
        gr, gc, _ = bufs[c % 2]

        def e_body(e16, acc_in):
            lane = e16 * L + lax.iota(jnp.int32, L)
            za = jnp.zeros((2 * L,), jnp.bfloat16)
            zb = jnp.zeros((2 * L,), jnp.bfloat16)
            for p in range(HP):
                pv = jnp.full((L,), p, jnp.int32)
                a = plsc.bitcast(plsc.load_gather(gr, [lane, pv]),
                                 jnp.bfloat16)
                b = plsc.bitcast(plsc.load_gather(gc, [lane, pv]),
                                 jnp.bfloat16)
                d = a - b
                if p < HP // 2:
                    za = za + d * d
                else:
                    zb = zb + d * d
            s = _halves(za) + _halves(zb) + jnp.float32(1e-12)
            dist = _sqrt16(s)
            wv = w_v[pl.ds(c * CHUNK + e16 * L, L)]
            return acc_in + dist * wv

        return lax.fori_loop(0, CHUNK // L, e_body, acc)

    pend = fire(0)
    acc = jnp.zeros((L,), jnp.float32)
    for c in range(CPW):
        nxt = fire(c + 1) if c + 1 < CPW else []
        for cp in pend:
            cp.wait()
        pend = nxt
        acc = compute(c, acc)

    acc_v[...] = acc
    pltpu.sync_copy(acc_v, out_hbm.at[wid])


def _comb_body(x_ref, y_ref, w_ref, b_ref, p_ref, o_ref):
    xh = (jnp.dot(y_ref[...], w_ref[...], preferred_element_type=jnp.float32)
          + b_ref[...])
    r = x_ref[...] - xh
    l1 = jnp.sum(r * r) * jnp.float32(1.0 / (N * D))
    l2 = jnp.sum(p_ref[...]) * jnp.float32(1.0 / E)
    o_ref[0, 0] = l1 + l2


def _combine(x, Y, W_dec, b_dec, parts):
    return pl.pallas_call(
        _comb_body,
        out_shape=jax.ShapeDtypeStruct((1, 1), jnp.float32),
        out_specs=pl.BlockSpec(memory_space=pltpu.SMEM),
    )(x, Y, W_dec, b_dec.reshape(1, D), parts)


def kernel(x, edge_index, edge_weight, W_enc, b_enc, W_dec, b_dec):
    pad = E_PAD - E
    row = jnp.concatenate([edge_index[0], jnp.zeros((pad,), jnp.int32)])
    col = jnp.concatenate([edge_index[1], jnp.zeros((pad,), jnp.int32)])
    w = jnp.concatenate([edge_weight, jnp.zeros((pad,), jnp.float32)])
    row2d = row.reshape(E_PAD // 128, 128)
    col2d = col.reshape(E_PAD // 128, 128)

    Y = _encode(x, W_enc, b_enc)
    Yp = lax.bitcast_convert_type(
        Y.astype(jnp.bfloat16).reshape(N, HP, 2), jnp.int32)
    parts = _sc_edge_partials(row2d, col2d, w, Yp)
    out = _combine(x, Y, W_dec, b_dec, parts)
    return out[0, 0]


# speedup vs baseline: 14.5862x; 1.1618x over previous
import functools

import jax
import jax.numpy as jnp
from jax import lax
from jax.experimental import pallas as pl
from jax.experimental.pallas import tpu as pltpu
from jax.experimental.pallas import tpu_sc as plsc

N = 10000
D = 128
E = 320000
H = 32
HP = H // 2

NC = 2
NS = 16
L = 16
NW = NC * NS

EPW = E // NW
CHUNK = 1024
NFULL = EPW // CHUNK
TAIL = EPW - NFULL * CHUNK

GRID = 10
BN = N // GRID


def _enc_body(x_ref, we_ref, wo_ref, be_ref, bo_ref, ye_ref, yo_ref, yp_ref):
    ye = jnp.tanh(
        jnp.dot(x_ref[...], we_ref[...], preferred_element_type=jnp.float32)
        + be_ref[...])
    yo = jnp.tanh(
        jnp.dot(x_ref[...], wo_ref[...], preferred_element_type=jnp.float32)
        + bo_ref[...])
    ye_ref[...] = ye
    yo_ref[...] = yo
    pe = lax.bitcast_convert_type(
        ye.astype(jnp.bfloat16), jnp.uint16).astype(jnp.uint32)
    po = lax.bitcast_convert_type(
        yo.astype(jnp.bfloat16), jnp.uint16).astype(jnp.uint32)
    yp_ref[...] = (pe | (po << 16)).astype(jnp.int32)


def _encode(x, W_enc, b_enc):
    return pl.pallas_call(
        _enc_body,
        grid=(GRID,),
        in_specs=[
            pl.BlockSpec((BN, D), lambda i: (i, 0)),
            pl.BlockSpec((D, HP), lambda i: (0, 0)),
            pl.BlockSpec((D, HP), lambda i: (0, 0)),
            pl.BlockSpec((1, HP), lambda i: (0, 0)),
            pl.BlockSpec((1, HP), lambda i: (0, 0)),
        ],
        out_specs=[
            pl.BlockSpec((BN, HP), lambda i: (i, 0)),
            pl.BlockSpec((BN, HP), lambda i: (i, 0)),
            pl.BlockSpec((BN, HP), lambda i: (i, 0)),
        ],
        out_shape=[
            jax.ShapeDtypeStruct((N, HP), jnp.float32),
            jax.ShapeDtypeStruct((N, HP), jnp.float32),
            jax.ShapeDtypeStruct((N, HP), jnp.int32),
        ],
    )(x, W_enc[:, 0::2], W_enc[:, 1::2],
      b_enc[0::2].reshape(1, HP), b_enc[1::2].reshape(1, HP))


def _sqrt16(x):
    i = plsc.bitcast(x, jnp.int32)
    i = jnp.int32(0x5F3759DF) - (i >> 1)
    y = plsc.bitcast(i, jnp.float32)
    for _ in range(3):
        y = y * (jnp.float32(1.5) - jnp.float32(0.5) * x * y * y)
    return x * y


def _halves(acc_bf):
    ai = plsc.bitcast(acc_bf, jnp.int32)
    lo = plsc.bitcast(lax.shift_left(ai, 16), jnp.float32)
    hi = plsc.bitcast(jnp.bitwise_and(ai, jnp.int32(-65536)), jnp.float32)
    return lo + hi


_mesh = plsc.VectorSubcoreMesh(core_axis_name="c", subcore_axis_name="s")


@functools.partial(
    pl.kernel,
    out_type=jax.ShapeDtypeStruct((NW, L), jnp.float32),
    mesh=_mesh,
    compiler_params=pltpu.CompilerParams(
        needs_layout_passes=False, use_tc_tiling_on_sc=False),
    scratch_types=[
        pltpu.VMEM_SHARED((N, HP), jnp.int32),
        pltpu.VMEM((EPW,), jnp.int32),
        pltpu.VMEM((EPW,), jnp.int32),
        pltpu.VMEM((EPW,), jnp.float32),
        pltpu.VMEM((CHUNK, HP), jnp.int32),
        pltpu.VMEM((CHUNK, HP), jnp.int32),
        pltpu.VMEM((CHUNK, HP), jnp.int32),
        pltpu.VMEM((CHUNK, HP), jnp.int32),
        pltpu.VMEM((L,), jnp.float32),
        pltpu.SemaphoreType.DMA,
        pltpu.SemaphoreType.DMA,
    ],
)
def _sc_edge_partials(ei_hbm, w_hbm, yp_hbm, out_hbm,
                      ysh, idxr_v, idxc_v, w_v,
                      gra_v, gca_v, grb_v, gcb_v, acc_v, sema, semb):
    cid = lax.axis_index("c")
    sid = lax.axis_index("s")
    wid = sid * NC + cid

    @pl.when(sid == 0)
    def _stage():
        pltpu.sync_copy(yp_hbm, ysh)

    base = wid * EPW
    pltpu.sync_copy(ei_hbm.at[0, pl.ds(base, EPW)], idxr_v)
    pltpu.sync_copy(ei_hbm.at[1, pl.ds(base, EPW)], idxc_v)
    pltpu.sync_copy(w_hbm.at[pl.ds(base, EPW)], w_v)
    plsc.subcore_barrier()

    bufs = [(gra_v, gca_v, sema), (grb_v, gcb_v, semb)]

    def fire(c):
        gr, gc, sem = bufs[c % 2]
        nrows = CHUNK if c < NFULL else TAIL
        cps = []
        for r0 in range(0, nrows, 128):
            n = min(128, nrows - r0)
            cps.append(pltpu.async_copy(
                ysh.at[idxr_v.at[pl.ds(c * CHUNK + r0, n)]],
                gr.at[pl.ds(r0, n)], sem))
            cps.append(pltpu.async_copy(
                ysh.at[idxc_v.at[pl.ds(c * CHUNK + r0, n)]],
                gc.at[pl.ds(r0, n)], sem))
        return cps

    def compute(c, acc):
        gr, gc, _ = bufs[c % 2]
        n16 = (CHUNK if c < NFULL else TAIL) // L

        def e_body(e16, acc_in):
            lane = e16 * L + lax.iota(jnp.int32, L)
            za = jnp.zeros((2 * L,), jnp.bfloat16)
            zb = jnp.zeros((2 * L,), jnp.bfloat16)
            for p in range(HP):
                pv = jnp.full((L,), p, jnp.int32)
                a = plsc.bitcast(plsc.load_gather(gr, [lane, pv]),
                                 jnp.bfloat16)
                b = plsc.bitcast(plsc.load_gather(gc, [lane, pv]),
                                 jnp.bfloat16)
                d = a - b
                if p < HP // 2:
                    za = za + d * d
                else:
                    zb = zb + d * d
            s = _halves(za) + _halves(zb) + jnp.float32(1e-12)
            dist = _sqrt16(s)
            wv = w_v[pl.ds(c * CHUNK + e16 * L, L)]
            return acc_in + dist * wv

        return lax.fori_loop(0, n16, e_body, acc)

    pend = fire(0)
    acc = jnp.zeros((L,), jnp.float32)
    for c in range(NFULL + 1):
        nxt = fire(c + 1) if c + 1 < NFULL + 1 else []
        for cp in pend:
            cp.wait()
        pend = nxt
        acc = compute(c, acc)

    acc_v[...] = acc
    pltpu.sync_copy(acc_v, out_hbm.at[wid])


def _loss1_body(x_ref, ye_ref, yo_ref, we_ref, wo_ref, b_ref, o_ref):
    xh = (jnp.dot(ye_ref[...], we_ref[...], preferred_element_type=jnp.float32)
          + jnp.dot(yo_ref[...], wo_ref[...], preferred_element_type=jnp.float32)
          + b_ref[...])
    r = x_ref[...] - xh
    part = jnp.sum(r * r)

    @pl.when(pl.program_id(0) == 0)
    def _init():
        o_ref[0, 0] = 0.0

    o_ref[0, 0] += part


def _loss1(x, ye, yo, W_dec, b_dec):
    return pl.pallas_call(
        _loss1_body,
        grid=(GRID,),
        in_specs=[
            pl.BlockSpec((BN, D), lambda i: (i, 0)),
            pl.BlockSpec((BN, HP), lambda i: (i, 0)),
            pl.BlockSpec((BN, HP), lambda i: (i, 0)),
            pl.BlockSpec((HP, D), lambda i: (0, 0)),
            pl.BlockSpec((HP, D), lambda i: (0, 0)),
            pl.BlockSpec((1, D), lambda i: (0, 0)),
        ],
        out_specs=pl.BlockSpec(memory_space=pltpu.SMEM),
        out_shape=jax.ShapeDtypeStruct((1, 1), jnp.float32),
    )(x, ye, yo, W_dec[0::2, :], W_dec[1::2, :], b_dec.reshape(1, D))


def _comb_body(l1_ref, p_ref, o_ref):
    l1 = l1_ref[0, 0] * jnp.float32(1.0 / (N * D))
    l2 = jnp.sum(p_ref[...]) * jnp.float32(1.0 / E)
    o_ref[0, 0] = l1 + l2


def _combine(l1, parts):
    return pl.pallas_call(
        _comb_body,
        in_specs=[
            pl.BlockSpec(memory_space=pltpu.SMEM),
            pl.BlockSpec(memory_space=pltpu.VMEM),
        ],
        out_specs=pl.BlockSpec(memory_space=pltpu.SMEM),
        out_shape=jax.ShapeDtypeStruct((1, 1), jnp.float32),
    )(l1, parts)


def kernel(x, edge_index, edge_weight, W_enc, b_enc, W_dec, b_dec):
    ye, yo, yp = _encode(x, W_enc, b_enc)
    parts = _sc_edge_partials(edge_index, edge_weight, yp)
    l1 = _loss1(x, ye, yo, W_dec, b_dec)
    out = _combine(l1, parts)
    return out[0, 0]


# speedup vs baseline: 14.7883x; 1.0139x over previous
import functools

import jax
import jax.numpy as jnp
from jax import lax
from jax.experimental import pallas as pl
from jax.experimental.pallas import tpu as pltpu
from jax.experimental.pallas import tpu_sc as plsc

N = 10000
D = 128
E = 320000
H = 32
HP = H // 2

NC = 2
NS = 16
L = 16
NW = NC * NS

EPW = E // NW
CHUNK = 1024
NFULL = EPW // CHUNK
TAIL = EPW - NFULL * CHUNK

GRID = 5
BN = N // GRID


def _enc_body(x_ref, we_ref, wo_ref, be_ref, bo_ref, ye_ref, yo_ref, yp_ref):
    ye = jnp.tanh(
        jnp.dot(x_ref[...], we_ref[...], preferred_element_type=jnp.float32)
        + be_ref[...])
    yo = jnp.tanh(
        jnp.dot(x_ref[...], wo_ref[...], preferred_element_type=jnp.float32)
        + bo_ref[...])
    ye_ref[...] = ye
    yo_ref[...] = yo
    pe = lax.bitcast_convert_type(
        ye.astype(jnp.bfloat16), jnp.uint16).astype(jnp.uint32)
    po = lax.bitcast_convert_type(
        yo.astype(jnp.bfloat16), jnp.uint16).astype(jnp.uint32)
    yp_ref[...] = (pe | (po << 16)).astype(jnp.int32)


def _encode(x, W_enc, b_enc):
    return pl.pallas_call(
        _enc_body,
        out_shape=[
            jax.ShapeDtypeStruct((N, HP), jnp.float32),
            jax.ShapeDtypeStruct((N, HP), jnp.float32),
            jax.ShapeDtypeStruct((N, HP), jnp.int32),
        ],
    )(x, W_enc[:, 0::2], W_enc[:, 1::2],
      b_enc[0::2].reshape(1, HP), b_enc[1::2].reshape(1, HP))


def _sqrt16(x):
    i = plsc.bitcast(x, jnp.int32)
    i = jnp.int32(0x5F3759DF) - (i >> 1)
    y = plsc.bitcast(i, jnp.float32)
    for _ in range(2):
        y = y * (jnp.float32(1.5) - jnp.float32(0.5) * x * y * y)
    return x * y


def _halves(acc_bf):
    ai = plsc.bitcast(acc_bf, jnp.int32)
    lo = plsc.bitcast(lax.shift_left(ai, 16), jnp.float32)
    hi = plsc.bitcast(jnp.bitwise_and(ai, jnp.int32(-65536)), jnp.float32)
    return lo + hi


_mesh = plsc.VectorSubcoreMesh(core_axis_name="c", subcore_axis_name="s")


@functools.partial(
    pl.kernel,
    out_type=jax.ShapeDtypeStruct((NW, L), jnp.float32),
    mesh=_mesh,
    compiler_params=pltpu.CompilerParams(
        needs_layout_passes=False, use_tc_tiling_on_sc=False),
    scratch_types=[
        pltpu.VMEM_SHARED((N, HP), jnp.int32),
        pltpu.VMEM((EPW,), jnp.int32),
        pltpu.VMEM((EPW,), jnp.int32),
        pltpu.VMEM((EPW,), jnp.float32),
        pltpu.VMEM((CHUNK, HP), jnp.int32),
        pltpu.VMEM((CHUNK, HP), jnp.int32),
        pltpu.VMEM((CHUNK, HP), jnp.int32),
        pltpu.VMEM((CHUNK, HP), jnp.int32),
        pltpu.VMEM((L,), jnp.float32),
        pltpu.SemaphoreType.DMA,
        pltpu.SemaphoreType.DMA,
    ],
)
def _sc_edge_partials(ei_hbm, w_hbm, yp_hbm, out_hbm,
                      ysh, idxr_v, idxc_v, w_v,
                      gra_v, gca_v, grb_v, gcb_v, acc_v, sema, semb):
    cid = lax.axis_index("c")
    sid = lax.axis_index("s")
    wid = sid * NC + cid

    @pl.when(sid == 0)
    def _stage():
        pltpu.sync_copy(yp_hbm, ysh)

    base = wid * EPW
    pltpu.sync_copy(ei_hbm.at[0, pl.ds(base, EPW)], idxr_v)
    pltpu.sync_copy(ei_hbm.at[1, pl.ds(base, EPW)], idxc_v)
    pltpu.sync_copy(w_hbm.at[pl.ds(base, EPW)], w_v)
    plsc.subcore_barrier()

    bufs = [(gra_v, gca_v, sema), (grb_v, gcb_v, semb)]

    def fire(c):
        gr, gc, sem = bufs[c % 2]
        nrows = CHUNK if c < NFULL else TAIL
        cps = []
        for r0 in range(0, nrows, 128):
            n = min(128, nrows - r0)
            cps.append(pltpu.async_copy(
                ysh.at[idxr_v.at[pl.ds(c * CHUNK + r0, n)]],
                gr.at[pl.ds(r0, n)], sem))
            cps.append(pltpu.async_copy(
                ysh.at[idxc_v.at[pl.ds(c * CHUNK + r0, n)]],
                gc.at[pl.ds(r0, n)], sem))
        return cps

    def compute(c, acc):
        gr, gc, _ = bufs[c % 2]
        n16 = (CHUNK if c < NFULL else TAIL) // L

        def e_body(e16, acc_in):
            lane = e16 * L + lax.iota(jnp.int32, L)
            za = jnp.zeros((2 * L,), jnp.bfloat16)
            zb = jnp.zeros((2 * L,), jnp.bfloat16)
            for p in range(HP):
                pv = jnp.full((L,), p, jnp.int32)
                a = plsc.bitcast(plsc.load_gather(gr, [lane, pv]),
                                 jnp.bfloat16)
                b = plsc.bitcast(plsc.load_gather(gc, [lane, pv]),
                                 jnp.bfloat16)
                d = a - b
                if p < HP // 2:
                    za = za + d * d
                else:
                    zb = zb + d * d
            s = _halves(za) + _halves(zb) + jnp.float32(1e-12)
            dist = _sqrt16(s)
            wv = w_v[pl.ds(c * CHUNK + e16 * L, L)]
            return acc_in + dist * wv

        return lax.fori_loop(0, n16, e_body, acc)

    pend = fire(0)
    acc = jnp.zeros((L,), jnp.float32)
    for c in range(NFULL + 1):
        nxt = fire(c + 1) if c + 1 < NFULL + 1 else []
        for cp in pend:
            cp.wait()
        pend = nxt
        acc = compute(c, acc)

    acc_v[...] = acc
    pltpu.sync_copy(acc_v, out_hbm.at[wid])


def _loss1_body(x_ref, ye_ref, yo_ref, we_ref, wo_ref, b_ref, o_ref):
    xh = (jnp.dot(ye_ref[...], we_ref[...], preferred_element_type=jnp.float32)
          + jnp.dot(yo_ref[...], wo_ref[...], preferred_element_type=jnp.float32)
          + b_ref[...])
    r = x_ref[...] - xh
    part = jnp.sum(r * r)

    @pl.when(pl.program_id(0) == 0)
    def _init():
        o_ref[0, 0] = 0.0

    o_ref[0, 0] += part


def _loss1(x, ye, yo, W_dec, b_dec):
    return pl.pallas_call(
        _loss1_body,
        grid=(GRID,),
        in_specs=[
            pl.BlockSpec((BN, D), lambda i: (i, 0)),
            pl.BlockSpec((BN, HP), lambda i: (i, 0)),
            pl.BlockSpec((BN, HP), lambda i: (i, 0)),
            pl.BlockSpec((HP, D), lambda i: (0, 0)),
            pl.BlockSpec((HP, D), lambda i: (0, 0)),
            pl.BlockSpec((1, D), lambda i: (0, 0)),
        ],
        out_specs=pl.BlockSpec(memory_space=pltpu.SMEM),
        out_shape=jax.ShapeDtypeStruct((1, 1), jnp.float32),
    )(x, ye, yo, W_dec[0::2, :], W_dec[1::2, :], b_dec.reshape(1, D))


def _comb_body(l1_ref, p_ref, o_ref):
    l1 = l1_ref[0, 0] * jnp.float32(1.0 / (N * D))
    l2 = jnp.sum(p_ref[...]) * jnp.float32(1.0 / E)
    o_ref[0, 0] = l1 + l2


def _combine(l1, parts):
    return pl.pallas_call(
        _comb_body,
        in_specs=[
            pl.BlockSpec(memory_space=pltpu.SMEM),
            pl.BlockSpec(memory_space=pltpu.VMEM),
        ],
        out_specs=pl.BlockSpec(memory_space=pltpu.SMEM),
        out_shape=jax.ShapeDtypeStruct((1, 1), jnp.float32),
    )(l1, parts)


def kernel(x, edge_index, edge_weight, W_enc, b_enc, W_dec, b_dec):
    ye, yo, yp = _encode(x, W_enc, b_enc)
    parts = _sc_edge_partials(edge_index, edge_weight, yp)
    l1 = _loss1(x, ye, yo, W_dec, b_dec)
    out = _combine(l1, parts)
    return out[0, 0]


# speedup vs baseline: 23.2033x; 1.5690x over previous
import functools

import jax
import jax.numpy as jnp
from jax import lax
from jax.experimental import pallas as pl
from jax.experimental.pallas import tpu as pltpu
from jax.experimental.pallas import tpu_sc as plsc

N = 10000
D = 128
E = 320000
H = 32
HQ = H // 4

NC = 2
NS = 16
L = 16
NW = NC * NS

EPW = E // NW
CHUNK = 1024
NFULL = EPW // CHUNK
TAIL = EPW - NFULL * CHUNK

GRID = 5
BN = N // GRID


def _enc_body(x_ref, w_ref, b_ref, y0_ref, y1_ref, y2_ref, y3_ref, yp_ref):
    yq = []
    for q in range(4):
        yq.append(jnp.tanh(
            jnp.dot(x_ref[...], w_ref[..., q * HQ:(q + 1) * HQ],
                    preferred_element_type=jnp.float32)
            + b_ref[..., q * HQ:(q + 1) * HQ]))
    for q, ref in enumerate((y0_ref, y1_ref, y2_ref, y3_ref)):
        ref[...] = yq[q]
    pq = [lax.bitcast_convert_type(
        y.astype(jnp.float8_e4m3fn), jnp.uint8).astype(jnp.uint32)
        for y in yq]
    yp_ref[...] = (pq[0] | (pq[1] << 8) | (pq[2] << 16)
                   | (pq[3] << 24)).astype(jnp.int32)


def _encode(x, W_enc, b_enc):
    return pl.pallas_call(
        _enc_body,
        out_shape=[
            jax.ShapeDtypeStruct((N, HQ), jnp.float32),
            jax.ShapeDtypeStruct((N, HQ), jnp.float32),
            jax.ShapeDtypeStruct((N, HQ), jnp.float32),
            jax.ShapeDtypeStruct((N, HQ), jnp.float32),
            jax.ShapeDtypeStruct((N, HQ), jnp.int32),
        ],
    )(x, W_enc, b_enc.reshape(1, H))


def _sqrt16(x):
    i = plsc.bitcast(x, jnp.int32)
    i = jnp.int32(0x5F3759DF) - (i >> 1)
    y = plsc.bitcast(i, jnp.float32)
    for _ in range(2):
        y = y * (jnp.float32(1.5) - jnp.float32(0.5) * x * y * y)
    return x * y


def _halves(acc_bf):
    ai = plsc.bitcast(acc_bf, jnp.int32)
    lo = plsc.bitcast(lax.shift_left(ai, 16), jnp.float32)
    hi = plsc.bitcast(jnp.bitwise_and(ai, jnp.int32(-65536)), jnp.float32)
    return lo + hi


_mesh = plsc.VectorSubcoreMesh(core_axis_name="c", subcore_axis_name="s")


@functools.partial(
    pl.kernel,
    out_type=jax.ShapeDtypeStruct((NW, L), jnp.float32),
    mesh=_mesh,
    compiler_params=pltpu.CompilerParams(
        needs_layout_passes=False, use_tc_tiling_on_sc=False),
    scratch_types=[
        pltpu.VMEM_SHARED((N, HQ), jnp.int32),
        pltpu.VMEM((EPW,), jnp.int32),
        pltpu.VMEM((EPW,), jnp.int32),
        pltpu.VMEM((EPW,), jnp.float32),
        pltpu.VMEM((CHUNK, HQ), jnp.int32),
        pltpu.VMEM((CHUNK, HQ), jnp.int32),
        pltpu.VMEM((CHUNK, HQ), jnp.int32),
        pltpu.VMEM((CHUNK, HQ), jnp.int32),
        pltpu.VMEM((L,), jnp.float32),
        pltpu.SemaphoreType.DMA,
        pltpu.SemaphoreType.DMA,
    ],
)
def _sc_edge_partials(ei_hbm, w_hbm, yp_hbm, out_hbm,
                      ysh, idxr_v, idxc_v, w_v,
                      gra_v, gca_v, grb_v, gcb_v, acc_v, sema, semb):
    cid = lax.axis_index("c")
    sid = lax.axis_index("s")
    wid = sid * NC + cid

    @pl.when(sid == 0)
    def _stage():
        pltpu.sync_copy(yp_hbm, ysh)

    base = wid * EPW
    pltpu.sync_copy(ei_hbm.at[0, pl.ds(base, EPW)], idxr_v)
    pltpu.sync_copy(ei_hbm.at[1, pl.ds(base, EPW)], idxc_v)
    pltpu.sync_copy(w_hbm.at[pl.ds(base, EPW)], w_v)
    plsc.subcore_barrier()

    bufs = [(gra_v, gca_v, sema), (grb_v, gcb_v, semb)]

    def fire(c):
        gr, gc, sem = bufs[c % 2]
        nrows = CHUNK if c < NFULL else TAIL
        cps = []
        for r0 in range(0, nrows, 128):
            n = min(128, nrows - r0)
            cps.append(pltpu.async_copy(
                ysh.at[idxr_v.at[pl.ds(c * CHUNK + r0, n)]],
                gr.at[pl.ds(r0, n)], sem))
            cps.append(pltpu.async_copy(
                ysh.at[idxc_v.at[pl.ds(c * CHUNK + r0, n)]],
                gc.at[pl.ds(r0, n)], sem))
        return cps

    def compute(c, acc):
        gr, gc, _ = bufs[c % 2]
        n16 = (CHUNK if c < NFULL else TAIL) // L

        def e_body(e16, acc_in):
            lane = e16 * L + lax.iota(jnp.int32, L)
            za = jnp.zeros((2 * L,), jnp.bfloat16)
            zb = jnp.zeros((2 * L,), jnp.bfloat16)
            for p in range(HQ):
                pv = jnp.full((L,), p, jnp.int32)
                a8 = plsc.bitcast(plsc.load_gather(gr, [lane, pv]),
                                  jnp.float8_e4m3fn)
                b8 = plsc.bitcast(plsc.load_gather(gc, [lane, pv]),
                                  jnp.float8_e4m3fn)
                a_lo, a_hi = plsc.unpack(
                    a8, format=plsc.PackFormat.INTERLEAVED,
                    preferred_element_type=jnp.bfloat16)
                b_lo, b_hi = plsc.unpack(
                    b8, format=plsc.PackFormat.INTERLEAVED,
                    preferred_element_type=jnp.bfloat16)
                d_lo = a_lo - b_lo
                d_hi = a_hi - b_hi
                za = za + d_lo * d_lo
                zb = zb + d_hi * d_hi
            s = _halves(za) + _halves(zb) + jnp.float32(1e-12)
            dist = _sqrt16(s)
            wv = w_v[pl.ds(c * CHUNK + e16 * L, L)]
            return acc_in + dist * wv

        return lax.fori_loop(0, n16, e_body, acc)

    pend = fire(0)
    acc = jnp.zeros((L,), jnp.float32)
    for c in range(NFULL + 1):
        nxt = fire(c + 1) if c + 1 < NFULL + 1 else []
        for cp in pend:
            cp.wait()
        pend = nxt
        acc = compute(c, acc)

    acc_v[...] = acc
    pltpu.sync_copy(acc_v, out_hbm.at[wid])


def _loss1_body(x_ref, y0_ref, y1_ref, y2_ref, y3_ref, w_ref, b_ref, o_ref):
    xh = b_ref[...]
    for q, ref in enumerate((y0_ref, y1_ref, y2_ref, y3_ref)):
        xh = xh + jnp.dot(ref[...], w_ref[q * HQ:(q + 1) * HQ, :],
                          preferred_element_type=jnp.float32)
    r = x_ref[...] - xh
    part = jnp.sum(r * r)

    @pl.when(pl.program_id(0) == 0)
    def _init():
        o_ref[0, 0] = 0.0

    o_ref[0, 0] += part


def _loss1(x, yq, W_dec, b_dec):
    return pl.pallas_call(
        _loss1_body,
        grid=(GRID,),
        in_specs=[
            pl.BlockSpec((BN, D), lambda i: (i, 0)),
            pl.BlockSpec((BN, HQ), lambda i: (i, 0)),
            pl.BlockSpec((BN, HQ), lambda i: (i, 0)),
            pl.BlockSpec((BN, HQ), lambda i: (i, 0)),
            pl.BlockSpec((BN, HQ), lambda i: (i, 0)),
            pl.BlockSpec((H, D), lambda i: (0, 0)),
            pl.BlockSpec((1, D), lambda i: (0, 0)),
        ],
        out_specs=pl.BlockSpec(memory_space=pltpu.SMEM),
        out_shape=jax.ShapeDtypeStruct((1, 1), jnp.float32),
    )(x, *yq, W_dec, b_dec.reshape(1, D))


def _comb_body(l1_ref, p_ref, o_ref):
    l1 = l1_ref[0, 0] * jnp.float32(1.0 / (N * D))
    l2 = jnp.sum(p_ref[...]) * jnp.float32(1.0 / E)
    o_ref[0, 0] = l1 + l2


def _combine(l1, parts):
    return pl.pallas_call(
        _comb_body,
        in_specs=[
            pl.BlockSpec(memory_space=pltpu.SMEM),
            pl.BlockSpec(memory_space=pltpu.VMEM),
        ],
        out_specs=pl.BlockSpec(memory_space=pltpu.SMEM),
        out_shape=jax.ShapeDtypeStruct((1, 1), jnp.float32),
    )(l1, parts)


def kernel(x, edge_index, edge_weight, W_enc, b_enc, W_dec, b_dec):
    perm = jnp.arange(H).reshape(HQ, 4).T.reshape(H)
    Wp = W_enc[:, perm]
    bp = b_enc[perm]
    y0, y1, y2, y3, yp = _encode(x, Wp, bp)
    parts = _sc_edge_partials(edge_index, edge_weight, yp)
    l1 = _loss1(x, (y0, y1, y2, y3), W_dec[perm, :], b_dec)
    out = _combine(l1, parts)
    return out[0, 0]


# speedup vs baseline: 25.9723x; 1.1193x over previous
import functools

import jax
import jax.numpy as jnp
from jax import lax
from jax.experimental import pallas as pl
from jax.experimental.pallas import tpu as pltpu
from jax.experimental.pallas import tpu_sc as plsc

N = 10000
D = 128
E = 320000
H = 32
HQ = H // 4

NC = 2
NS = 16
L = 16
NW = NC * NS

EPW = E // NW
CHUNK = 1024
NFULL = EPW // CHUNK
TAIL = EPW - NFULL * CHUNK

GRID = 5
BN = N // GRID


def _enc_body(x_ref, w_ref, b_ref, y_ref, yp_ref):
    y = jnp.tanh(
        jnp.dot(x_ref[...], w_ref[...], preferred_element_type=jnp.float32)
        + b_ref[...])
    y_ref[...] = y
    rows = lax.broadcasted_iota(jnp.int32, (H, HQ), 0)
    cols = lax.broadcasted_iota(jnp.int32, (H, HQ), 1)
    pq = []
    for q in range(4):
        sel = (rows == 4 * cols + q).astype(jnp.float32)
        yq = jnp.dot(y, sel, preferred_element_type=jnp.float32)
        pq.append(lax.bitcast_convert_type(
            yq.astype(jnp.float8_e4m3fn), jnp.uint8).astype(jnp.uint32))
    yp_ref[...] = (pq[0] | (pq[1] << 8) | (pq[2] << 16)
                   | (pq[3] << 24)).astype(jnp.int32)


def _encode(x, W_enc, b_enc):
    return pl.pallas_call(
        _enc_body,
        out_shape=[
            jax.ShapeDtypeStruct((N, H), jnp.float32),
            jax.ShapeDtypeStruct((N, HQ), jnp.int32),
        ],
    )(x, W_enc, b_enc.reshape(1, H))


def _sqrt16(x):
    i = plsc.bitcast(x, jnp.int32)
    i = jnp.int32(0x5F3759DF) - (i >> 1)
    y = plsc.bitcast(i, jnp.float32)
    for _ in range(2):
        y = y * (jnp.float32(1.5) - jnp.float32(0.5) * x * y * y)
    return x * y


def _halves(acc_bf):
    ai = plsc.bitcast(acc_bf, jnp.int32)
    lo = plsc.bitcast(lax.shift_left(ai, 16), jnp.float32)
    hi = plsc.bitcast(jnp.bitwise_and(ai, jnp.int32(-65536)), jnp.float32)
    return lo + hi


_mesh = plsc.VectorSubcoreMesh(core_axis_name="c", subcore_axis_name="s")


@functools.partial(
    pl.kernel,
    out_type=jax.ShapeDtypeStruct((NW * L,), jnp.float32),
    mesh=_mesh,
    compiler_params=pltpu.CompilerParams(
        needs_layout_passes=False, use_tc_tiling_on_sc=False),
    scratch_types=[
        pltpu.VMEM_SHARED((N, HQ), jnp.int32),
        pltpu.VMEM((EPW,), jnp.int32),
        pltpu.VMEM((EPW,), jnp.int32),
        pltpu.VMEM((EPW,), jnp.float32),
        pltpu.VMEM((CHUNK, HQ), jnp.int32),
        pltpu.VMEM((CHUNK, HQ), jnp.int32),
        pltpu.VMEM((CHUNK, HQ), jnp.int32),
        pltpu.VMEM((CHUNK, HQ), jnp.int32),
        pltpu.VMEM((L,), jnp.float32),
        pltpu.SemaphoreType.DMA,
        pltpu.SemaphoreType.DMA,
    ],
)
def _sc_edge_partials(ei_hbm, w_hbm, yp_hbm, out_hbm,
                      ysh, idxr_v, idxc_v, w_v,
                      gra_v, gca_v, grb_v, gcb_v, acc_v, sema, semb):
    cid = lax.axis_index("c")
    sid = lax.axis_index("s")
    wid = sid * NC + cid

    @pl.when(sid == 0)
    def _stage():
        pltpu.sync_copy(yp_hbm, ysh)

    base = wid * EPW
    pltpu.sync_copy(ei_hbm.at[0, pl.ds(base, EPW)], idxr_v)
    pltpu.sync_copy(ei_hbm.at[1, pl.ds(base, EPW)], idxc_v)
    pltpu.sync_copy(w_hbm.at[pl.ds(base, EPW)], w_v)
    plsc.subcore_barrier()

    bufs = [(gra_v, gca_v, sema), (grb_v, gcb_v, semb)]

    def fire(c):
        gr, gc, sem = bufs[c % 2]
        nrows = CHUNK if c < NFULL else TAIL
        cps = []
        for r0 in range(0, nrows, 128):
            n = min(128, nrows - r0)
            cps.append(pltpu.async_copy(
                ysh.at[idxr_v.at[pl.ds(c * CHUNK + r0, n)]],
                gr.at[pl.ds(r0, n)], sem))
            cps.append(pltpu.async_copy(
                ysh.at[idxc_v.at[pl.ds(c * CHUNK + r0, n)]],
                gc.at[pl.ds(r0, n)], sem))
        return cps

    def compute(c, acc):
        gr, gc, _ = bufs[c % 2]
        n16 = (CHUNK if c < NFULL else TAIL) // L

        def e_body(e16, acc_in):
            lane = e16 * L + lax.iota(jnp.int32, L)
            za = jnp.zeros((2 * L,), jnp.bfloat16)
            zb = jnp.zeros((2 * L,), jnp.bfloat16)
            for p in range(HQ):
                pv = jnp.full((L,), p, jnp.int32)
                a8 = plsc.bitcast(plsc.load_gather(gr, [lane, pv]),
                                  jnp.float8_e4m3fn)
                b8 = plsc.bitcast(plsc.load_gather(gc, [lane, pv]),
                                  jnp.float8_e4m3fn)
                a_lo, a_hi = plsc.unpack(
                    a8, format=plsc.PackFormat.INTERLEAVED,
                    preferred_element_type=jnp.bfloat16)
                b_lo, b_hi = plsc.unpack(
                    b8, format=plsc.PackFormat.INTERLEAVED,
                    preferred_element_type=jnp.bfloat16)
                d_lo = a_lo - b_lo
                d_hi = a_hi - b_hi
                za = za + d_lo * d_lo
                zb = zb + d_hi * d_hi
            s = _halves(za) + _halves(zb) + jnp.float32(1e-12)
            dist = _sqrt16(s)
            wv = w_v[pl.ds(c * CHUNK + e16 * L, L)]
            return acc_in + dist * wv

        return lax.fori_loop(0, n16, e_body, acc)

    pend = fire(0)
    acc = jnp.zeros((L,), jnp.float32)
    for c in range(NFULL + 1):
        nxt = fire(c + 1) if c + 1 < NFULL + 1 else []
        for cp in pend:
            cp.wait()
        pend = nxt
        acc = compute(c, acc)

    acc_v[...] = acc
    pltpu.sync_copy(acc_v, out_hbm.at[pl.ds(wid * L, L)])


def _loss1_body(x_ref, y_ref, w_ref, b_ref, o_ref):
    xh = (jnp.dot(y_ref[...], w_ref[...], preferred_element_type=jnp.float32)
          + b_ref[...])
    r = x_ref[...] - xh
    part = jnp.sum(r * r)

    @pl.when(pl.program_id(0) == 0)
    def _init():
        o_ref[0, 0] = 0.0

    o_ref[0, 0] += part


def _loss1(x, y, W_dec, b_dec):
    return pl.pallas_call(
        _loss1_body,
        grid=(GRID,),
        in_specs=[
            pl.BlockSpec((BN, D), lambda i: (i, 0)),
            pl.BlockSpec((BN, H), lambda i: (i, 0)),
            pl.BlockSpec((H, D), lambda i: (0, 0)),
            pl.BlockSpec((1, D), lambda i: (0, 0)),
        ],
        out_specs=pl.BlockSpec(memory_space=pltpu.SMEM),
        out_shape=jax.ShapeDtypeStruct((1, 1), jnp.float32),
    )(x, y, W_dec, b_dec.reshape(1, D))


def _comb_body(l1_ref, p_ref, o_ref):
    l1 = l1_ref[0, 0] * jnp.float32(1.0 / (N * D))
    l2 = jnp.sum(p_ref[...]) * jnp.float32(1.0 / E)
    o_ref[0, 0] = l1 + l2


def _combine(l1, parts):
    return pl.pallas_call(
        _comb_body,
        in_specs=[
            pl.BlockSpec(memory_space=pltpu.SMEM),
            pl.BlockSpec(memory_space=pltpu.VMEM),
        ],
        out_specs=pl.BlockSpec(memory_space=pltpu.SMEM),
        out_shape=jax.ShapeDtypeStruct((1, 1), jnp.float32),
    )(l1, parts)


def kernel(x, edge_index, edge_weight, W_enc, b_enc, W_dec, b_dec):
    y, yp = _encode(x, W_enc, b_enc)
    parts = _sc_edge_partials(edge_index, edge_weight, yp)
    l1 = _loss1(x, y, W_dec, b_dec)
    out = _combine(l1, parts)
    return out[0, 0]
